# baseline (speedup 1.0000x reference)
import numpy as np

import jax
import jax.numpy as jnp
from jax.experimental import pallas as pl
from jax.experimental.pallas import tpu as pltpu


BB = 1024


_U_ROW = np.zeros((2, 5, 6), np.float32)
for _u in range(2):
    for _di in range(5):
        _U_ROW[_u, _di, _u + _di] = 1.0
_V_COL = np.zeros((2, 12, 5, 28), np.float32)
for _v in range(2):
    for _jp in range(12):
        for _dj in range(5):
            _V_COL[_v, _jp, _dj, 2 * _jp + _v + _dj] = 1.0
_C2_SEL = np.zeros((160, 120), np.float32)
for _ci in range(10):
    for _w in range(12):
        _C2_SEL[_ci * 16 + _w, _ci * 12 + _w] = 1.0


def _fused_kernel(xb_ref, a2_ref, b1v_ref, w2c_ref, b2v_ref,
                  fc1w_ref, fc1b_ref, fc2w_ref, fc2b_ref, o_ref,
                  xp_ref, p1_ref):

    xp_ref[...] = xb_ref[...].T

    for p in range(12):
        win = xp_ref[p * 56:p * 56 + 168, :]
        r = jnp.dot(a2_ref[...], win,
                    preferred_element_type=jnp.float32)
        m = jnp.maximum(jnp.maximum(r[0:120], r[120:240]),
                        jnp.maximum(r[240:360], r[360:480]))
        p1_ref[p * 120:(p + 1) * 120, :] = jnp.maximum(
            m + b1v_ref[...], 0.0).astype(jnp.bfloat16)

    rmax = []
    for i in range(8):
        c2 = jnp.dot(w2c_ref[...], p1_ref[i * 120:i * 120 + 600, :],
                     preferred_element_type=jnp.float32)
        rmax.append(jnp.maximum(c2[0:80], c2[80:160]))

    b2v = b2v_ref[...]
    flat = jnp.concatenate(
        [jnp.maximum(jnp.maximum(rmax[2 * ip], rmax[2 * ip + 1]) + b2v, 0.0)
         for ip in range(4)], axis=0)

    h1 = jnp.maximum(
        jnp.dot(fc1w_ref[...], flat, preferred_element_type=jnp.float32)
        + fc1b_ref[...], 0.0)
    z = (jnp.dot(fc2w_ref[...], h1, preferred_element_type=jnp.float32)
         + fc2b_ref[...])

    zmax = jnp.max(z, axis=0, keepdims=True)
    s = z - zmax
    lse = jnp.log(jnp.sum(jnp.exp(s), axis=0, keepdims=True))
    o_ref[...] = (s - lse).T


def kernel(w1, b1, w2t, b2v, fc1_w, fc1_b, fc2_w, fc2_b, x_nchw):
    n = x_nchw.shape[0]
    npad = ((n + BB - 1) // BB) * BB

    xin = x_nchw[:, 0].reshape(n, 784).astype(jnp.bfloat16)
    if npad != n:
        xin = jnp.concatenate(
            [xin, jnp.zeros((npad - n, 784), xin.dtype)], axis=0)

    w1r = w1.reshape(10, 5, 5)
    t1 = jnp.einsum('cij,uie->ucej', w1r, jnp.asarray(_U_ROW))
    a2 = jnp.einsum('ucej,vpjk->uvcpek', t1,
                    jnp.asarray(_V_COL)).reshape(480, 168)
    b1v = jnp.repeat(b1, 12).reshape(120, 1)
    w2c = jnp.einsum('dms,st->mdt', w2t,
                     jnp.asarray(_C2_SEL)).reshape(160, 600)
    a2 = a2.astype(jnp.bfloat16)
    w2c = w2c.astype(jnp.bfloat16)

    out = pl.pallas_call(
        _fused_kernel,
        out_shape=jax.ShapeDtypeStruct((npad, 10), jnp.float32),
        grid=(npad // BB,),
        in_specs=[
            pl.BlockSpec((BB, 784), lambda i: (i, 0)),
            pl.BlockSpec((480, 168), lambda i: (0, 0)),
            pl.BlockSpec((120, 1), lambda i: (0, 0)),
            pl.BlockSpec((160, 600), lambda i: (0, 0)),
            pl.BlockSpec((80, 1), lambda i: (0, 0)),
            pl.BlockSpec((50, 320), lambda i: (0, 0)),
            pl.BlockSpec((50, 1), lambda i: (0, 0)),
            pl.BlockSpec((10, 50), lambda i: (0, 0)),
            pl.BlockSpec((10, 1), lambda i: (0, 0)),
        ],
        out_specs=pl.BlockSpec((BB, 10), lambda i: (i, 0)),
        scratch_shapes=[pltpu.VMEM((784, BB), jnp.bfloat16),
                        pltpu.VMEM((1440, BB), jnp.bfloat16)],
        compiler_params=pltpu.CompilerParams(
            dimension_semantics=("parallel",),
            vmem_limit_bytes=32 * 1024 * 1024),
    )(xin, a2, b1v, w2c, b2v, fc1_w, fc1_b, fc2_w, fc2_b)

    return out[:n]


# speedup vs baseline: 1.0274x; 1.0274x over previous
import numpy as np

import jax
import jax.numpy as jnp
from jax.experimental import pallas as pl
from jax.experimental.pallas import tpu as pltpu


BB = 2048


_U_ROW = np.zeros((2, 5, 6), np.float32)
for _u in range(2):
    for _di in range(5):
        _U_ROW[_u, _di, _u + _di] = 1.0
_V_COL = np.zeros((2, 12, 5, 28), np.float32)
for _v in range(2):
    for _jp in range(12):
        for _dj in range(5):
            _V_COL[_v, _jp, _dj, 2 * _jp + _v + _dj] = 1.0
_C2_SEL = np.zeros((160, 120), np.float32)
for _ci in range(10):
    for _w in range(12):
        _C2_SEL[_ci * 16 + _w, _ci * 12 + _w] = 1.0


def _fused_kernel(xb_ref, a2_ref, b1v_ref, w2c_ref, b2v_ref,
                  fc1w_ref, fc1b_ref, fc2w_ref, fc2b_ref, o_ref,
                  xp_ref, p1_ref):

    xp_ref[...] = xb_ref[...].T

    for p in range(12):
        win = xp_ref[p * 56:p * 56 + 168, :]
        r = jnp.dot(a2_ref[...], win,
                    preferred_element_type=jnp.float32)
        m = jnp.maximum(jnp.maximum(r[0:120], r[120:240]),
                        jnp.maximum(r[240:360], r[360:480]))
        p1_ref[p * 120:(p + 1) * 120, :] = jnp.maximum(
            m + b1v_ref[...], 0.0).astype(jnp.bfloat16)

    rmax = []
    for i in range(8):
        c2 = jnp.dot(w2c_ref[...], p1_ref[i * 120:i * 120 + 600, :],
                     preferred_element_type=jnp.float32)
        rmax.append(jnp.maximum(c2[0:80], c2[80:160]))

    b2v = b2v_ref[...]
    flat = jnp.concatenate(
        [jnp.maximum(jnp.maximum(rmax[2 * ip], rmax[2 * ip + 1]) + b2v, 0.0)
         for ip in range(4)], axis=0)

    h1 = jnp.maximum(
        jnp.dot(fc1w_ref[...], flat, preferred_element_type=jnp.float32)
        + fc1b_ref[...], 0.0)
    z = (jnp.dot(fc2w_ref[...], h1, preferred_element_type=jnp.float32)
         + fc2b_ref[...])

    zmax = jnp.max(z, axis=0, keepdims=True)
    s = z - zmax
    lse = jnp.log(jnp.sum(jnp.exp(s), axis=0, keepdims=True))
    o_ref[...] = (s - lse).T


def kernel(w1, b1, w2t, b2v, fc1_w, fc1_b, fc2_w, fc2_b, x_nchw):
    n = x_nchw.shape[0]
    npad = ((n + BB - 1) // BB) * BB

    xin = x_nchw[:, 0].reshape(n, 784).astype(jnp.bfloat16)
    if npad != n:
        xin = jnp.concatenate(
            [xin, jnp.zeros((npad - n, 784), xin.dtype)], axis=0)

    w1r = w1.reshape(10, 5, 5)
    t1 = jnp.einsum('cij,uie->ucej', w1r, jnp.asarray(_U_ROW))
    a2 = jnp.einsum('ucej,vpjk->uvcpek', t1,
                    jnp.asarray(_V_COL)).reshape(480, 168)
    b1v = jnp.repeat(b1, 12).reshape(120, 1)
    w2c = jnp.einsum('dms,st->mdt', w2t,
                     jnp.asarray(_C2_SEL)).reshape(160, 600)
    a2 = a2.astype(jnp.bfloat16)
    w2c = w2c.astype(jnp.bfloat16)

    out = pl.pallas_call(
        _fused_kernel,
        out_shape=jax.ShapeDtypeStruct((npad, 10), jnp.float32),
        grid=(npad // BB,),
        in_specs=[
            pl.BlockSpec((BB, 784), lambda i: (i, 0)),
            pl.BlockSpec((480, 168), lambda i: (0, 0)),
            pl.BlockSpec((120, 1), lambda i: (0, 0)),
            pl.BlockSpec((160, 600), lambda i: (0, 0)),
            pl.BlockSpec((80, 1), lambda i: (0, 0)),
            pl.BlockSpec((50, 320), lambda i: (0, 0)),
            pl.BlockSpec((50, 1), lambda i: (0, 0)),
            pl.BlockSpec((10, 50), lambda i: (0, 0)),
            pl.BlockSpec((10, 1), lambda i: (0, 0)),
        ],
        out_specs=pl.BlockSpec((BB, 10), lambda i: (i, 0)),
        scratch_shapes=[pltpu.VMEM((784, BB), jnp.bfloat16),
                        pltpu.VMEM((1440, BB), jnp.bfloat16)],
        compiler_params=pltpu.CompilerParams(
            dimension_semantics=("parallel",),
            vmem_limit_bytes=32 * 1024 * 1024),
    )(xin, a2, b1v, w2c, b2v, fc1_w, fc1_b, fc2_w, fc2_b)

    return out[:n]
